# baseline (speedup 1.0000x reference)
import functools

import jax
import jax.numpy as jnp
from jax import lax
from jax.experimental import pallas as pl
from jax.experimental.pallas import tpu as pltpu
from jax.experimental.pallas import tpu_sc as plsc

N = 10000
E = 320000
D = 128

NC = 2
NS = 16
NW = NC * NS
EW = E // NW
CHUNK = 40
ITERS = EW // CHUNK
NPAD = 10240
ROWS_PER_SUB = NPAD // NS

RING = 8

_mesh = plsc.VectorSubcoreMesh(core_axis_name="c", subcore_axis_name="s")


@functools.partial(
    pl.kernel,
    mesh=_mesh,
    out_type=jax.ShapeDtypeStruct((NC, NPAD, D), jnp.float32),
    scratch_types=(
        [
            pltpu.VMEM_SHARED((NPAD, D), jnp.float32),
            pltpu.VMEM((16, D), jnp.float32),
        ]
        + [pltpu.VMEM((CHUNK,), jnp.int32) for _ in range(RING)]
        + [pltpu.VMEM((CHUNK,), jnp.int32) for _ in range(RING)]
        + [pltpu.VMEM((CHUNK, D), jnp.float32) for _ in range(RING)]
        + [pltpu.SemaphoreType.DMA for _ in range(3 * RING + 1)]
    ),
)
def _sc_agg(x_hbm, src_hbm, dst_hbm, out_hbm, acc, zbuf, *rest):
    srcv = rest[0:RING]
    dstv = rest[RING:2 * RING]
    bufs = rest[2 * RING:3 * RING]
    gsem = rest[3 * RING:4 * RING]
    isem = rest[4 * RING:5 * RING]
    ssem = rest[5 * RING:6 * RING]
    zsem = rest[6 * RING]

    c = lax.axis_index("c")
    s = lax.axis_index("s")
    wid = s * NC + c
    base = wid * EW

    def fetch_idx(g, q):
        off = base + g * CHUNK
        pltpu.async_copy(src_hbm.at[pl.ds(off, CHUNK)], srcv[q], isem[q])
        pltpu.async_copy(dst_hbm.at[pl.ds(off, CHUNK)], dstv[q], isem[q])

    def wait_idx(q):
        pltpu.make_async_copy(src_hbm.at[pl.ds(0, CHUNK)], srcv[q],
                              isem[q]).wait()
        pltpu.make_async_copy(dst_hbm.at[pl.ds(0, CHUNK)], dstv[q],
                              isem[q]).wait()

    def start_gather(q):
        pltpu.async_copy(x_hbm.at[srcv[q]], bufs[q], gsem[q])

    def wait_gather(q):
        pltpu.make_async_copy(x_hbm.at[srcv[q]], bufs[q], gsem[q]).wait()

    def start_scatter(q):
        pltpu.async_copy(bufs[q], acc.at[dstv[q]], ssem[q], add=True)

    def wait_scatter(q):
        pltpu.make_async_copy(bufs[q], acc.at[dstv[q]], ssem[q]).wait()

    def visit(g, b, wait_sca, fetch, gather):
        wait_gather(b)
        start_scatter(b)
        if wait_sca:
            wait_scatter((b + 6) % RING)
        if fetch:
            fetch_idx(g + 6, (b + 6) % RING)
        if gather:
            wait_idx((b + 5) % RING)
            start_gather((b + 5) % RING)

    zero = jnp.zeros((16,), jnp.float32)
    for i in range(16):
        for j in range(D // 16):
            zbuf[i, pl.ds(j * 16, 16)] = zero

    row0 = s * ROWS_PER_SUB
    zcps = [pltpu.async_copy(zbuf, acc.at[pl.ds(row0 + r * 16, 16)], zsem)
            for r in range(ROWS_PER_SUB // 16)]

    for q in range(6):
        fetch_idx(q, q)
    for q in range(5):
        wait_idx(q)
        start_gather(q)
    for z in zcps:
        z.wait()
    plsc.subcore_barrier()

    visit(0, 0, wait_sca=False, fetch=True, gather=True)
    visit(1, 1, wait_sca=False, fetch=True, gather=True)

    def body(outer, carry):
        for j in range(RING):
            g = 2 + outer * RING + j
            visit(g, (2 + j) % RING, wait_sca=True, fetch=True, gather=True)
        return carry

    lax.fori_loop(0, (ITERS - 10) // RING, body, 0)

    visit(ITERS - 8, (ITERS - 8) % RING, wait_sca=True, fetch=True,
          gather=True)
    visit(ITERS - 7, (ITERS - 7) % RING, wait_sca=True, fetch=True,
          gather=True)
    visit(ITERS - 6, (ITERS - 6) % RING, wait_sca=True, fetch=False,
          gather=True)
    for t in range(5):
        visit(ITERS - 5 + t, (ITERS - 5 + t) % RING, wait_sca=True,
              fetch=False, gather=False)
    wait_scatter((ITERS - 2) % RING)
    wait_scatter((ITERS - 1) % RING)
    plsc.subcore_barrier()

    pltpu.sync_copy(acc.at[pl.ds(row0, ROWS_PER_SUB)],
                    out_hbm.at[c, pl.ds(row0, ROWS_PER_SUB)])


def _mlp_body(x_ref, a_ref, w1_ref, b1_ref, w2_ref, b2_ref, o_ref,
              *, relu_out):
    h = x_ref[...] + a_ref[0] + a_ref[1]
    h = jnp.dot(h, w1_ref[...], preferred_element_type=jnp.float32)
    h = jnp.maximum(h + b1_ref[...], 0.0)
    h = jnp.dot(h, w2_ref[...], preferred_element_type=jnp.float32)
    h = h + b2_ref[...]
    if relu_out:
        h = jnp.maximum(h, 0.0)
    o_ref[...] = h


def _mlp(x, parts, Wa, ba, Wb, bb, relu_out):
    BR = 1000
    row_spec = pl.BlockSpec((BR, D), lambda i: (i, 0))
    part_spec = pl.BlockSpec((NC, BR, D), lambda i: (0, i, 0))
    w_spec = pl.BlockSpec((D, D), lambda i: (0, 0))
    b_spec = pl.BlockSpec((1, D), lambda i: (0, 0))
    return pl.pallas_call(
        functools.partial(_mlp_body, relu_out=relu_out),
        grid=(N // BR,),
        in_specs=[row_spec, part_spec, w_spec, b_spec, w_spec, b_spec],
        out_specs=row_spec,
        out_shape=jax.ShapeDtypeStruct((N, D), jnp.float32),
    )(x, parts, Wa, ba.reshape(1, D), Wb, bb.reshape(1, D))


def kernel(x, edge_index, W1, b1, W2, b2, W3, b3, W4, b4):
    src = edge_index[0].astype(jnp.int32)
    dst = edge_index[1].astype(jnp.int32)
    p1 = _sc_agg(x, src, dst)
    h = _mlp(x, p1, W1, b1, W2, b2, relu_out=True)
    p2 = _sc_agg(h, src, dst)
    return _mlp(h, p2, W3, b3, W4, b4, relu_out=False)


# speedup vs baseline: 1.0260x; 1.0260x over previous
import functools

import jax
import jax.numpy as jnp
from jax import lax
from jax.experimental import pallas as pl
from jax.experimental.pallas import tpu as pltpu
from jax.experimental.pallas import tpu_sc as plsc

N = 10000
E = 320000
D = 128

NC = 2
NS = 16
NW = NC * NS
EW = E // NW
CHUNK = 40
ITERS = EW // CHUNK
NPAD = 10240
ROWS_PER_SUB = NPAD // NS

RING = 8

_mesh = plsc.VectorSubcoreMesh(core_axis_name="c", subcore_axis_name="s")


@functools.partial(
    pl.kernel,
    mesh=_mesh,
    out_type=jax.ShapeDtypeStruct((NC, NPAD, D), jnp.float32),
    scratch_types=(
        [
            pltpu.VMEM_SHARED((NPAD, D), jnp.float32),
            pltpu.VMEM((16, D), jnp.float32),
        ]
        + [pltpu.VMEM((CHUNK,), jnp.int32) for _ in range(RING)]
        + [pltpu.VMEM((CHUNK,), jnp.int32) for _ in range(RING)]
        + [pltpu.VMEM((CHUNK, D), jnp.float32) for _ in range(RING)]
        + [pltpu.SemaphoreType.DMA for _ in range(3 * RING + 1)]
    ),
)
def _sc_agg(x_hbm, src_hbm, dst_hbm, out_hbm, acc, zbuf, *rest):
    srcv = rest[0:RING]
    dstv = rest[RING:2 * RING]
    bufs = rest[2 * RING:3 * RING]
    gsem = rest[3 * RING:4 * RING]
    isem = rest[4 * RING:5 * RING]
    ssem = rest[5 * RING:6 * RING]
    zsem = rest[6 * RING]

    c = lax.axis_index("c")
    s = lax.axis_index("s")
    wid = s * NC + c
    base = wid * EW

    def fetch_idx(g, q):
        off = base + g * CHUNK
        pltpu.async_copy(src_hbm.at[pl.ds(off, CHUNK)], srcv[q], isem[q])
        pltpu.async_copy(dst_hbm.at[pl.ds(off, CHUNK)], dstv[q], isem[q])

    def wait_idx(q):
        pltpu.make_async_copy(src_hbm.at[pl.ds(0, CHUNK)], srcv[q],
                              isem[q]).wait()
        pltpu.make_async_copy(dst_hbm.at[pl.ds(0, CHUNK)], dstv[q],
                              isem[q]).wait()

    def start_gather(q):
        pltpu.async_copy(x_hbm.at[srcv[q]], bufs[q], gsem[q])

    def wait_gather(q):
        pltpu.make_async_copy(x_hbm.at[srcv[q]], bufs[q], gsem[q]).wait()

    def start_scatter(q):
        pltpu.async_copy(bufs[q], acc.at[dstv[q]], ssem[q], add=True)

    def wait_scatter(q):
        pltpu.make_async_copy(bufs[q], acc.at[dstv[q]], ssem[q]).wait()

    def visit(g, b, wait_sca, fetch, gather):
        wait_gather(b)
        start_scatter(b)
        if wait_sca:
            wait_scatter((b + 6) % RING)
        if fetch:
            fetch_idx(g + 6, (b + 6) % RING)
        if gather:
            wait_idx((b + 5) % RING)
            start_gather((b + 5) % RING)

    zero = jnp.zeros((16,), jnp.float32)
    for i in range(16):
        for j in range(D // 16):
            zbuf[i, pl.ds(j * 16, 16)] = zero

    row0 = s * ROWS_PER_SUB
    zcps = [pltpu.async_copy(zbuf, acc.at[pl.ds(row0 + r * 16, 16)], zsem)
            for r in range(ROWS_PER_SUB // 16)]

    for q in range(6):
        fetch_idx(q, q)
    for q in range(5):
        wait_idx(q)
        start_gather(q)
    for z in zcps:
        z.wait()
    plsc.subcore_barrier()

    visit(0, 0, wait_sca=False, fetch=True, gather=True)
    visit(1, 1, wait_sca=False, fetch=True, gather=True)

    def body(outer, carry):
        for j in range(RING):
            g = 2 + outer * RING + j
            visit(g, (2 + j) % RING, wait_sca=True, fetch=True, gather=True)
        return carry

    lax.fori_loop(0, (ITERS - 10) // RING, body, 0)

    visit(ITERS - 8, (ITERS - 8) % RING, wait_sca=True, fetch=True,
          gather=True)
    visit(ITERS - 7, (ITERS - 7) % RING, wait_sca=True, fetch=True,
          gather=True)
    visit(ITERS - 6, (ITERS - 6) % RING, wait_sca=True, fetch=False,
          gather=True)
    for t in range(5):
        visit(ITERS - 5 + t, (ITERS - 5 + t) % RING, wait_sca=True,
              fetch=False, gather=False)
    wait_scatter((ITERS - 2) % RING)
    wait_scatter((ITERS - 1) % RING)
    plsc.subcore_barrier()

    pltpu.sync_copy(acc.at[pl.ds(row0, ROWS_PER_SUB)],
                    out_hbm.at[c, pl.ds(row0, ROWS_PER_SUB)])


def _mlp_body(x_ref, a_ref, w1_ref, b1_ref, w2_ref, b2_ref, o_ref,
              *, relu_out):
    h = x_ref[...] + a_ref[0] + a_ref[1]
    h = jnp.dot(h, w1_ref[...], preferred_element_type=jnp.float32)
    h = jnp.maximum(h + b1_ref[...], 0.0)
    h = jnp.dot(h, w2_ref[...], preferred_element_type=jnp.float32)
    h = h + b2_ref[...]
    if relu_out:
        h = jnp.maximum(h, 0.0)
    o_ref[...] = h


def _mlp(x, parts, Wa, ba, Wb, bb, relu_out):
    BR = 2000
    row_spec = pl.BlockSpec((BR, D), lambda i: (i, 0))
    part_spec = pl.BlockSpec((NC, BR, D), lambda i: (0, i, 0))
    w_spec = pl.BlockSpec((D, D), lambda i: (0, 0))
    b_spec = pl.BlockSpec((1, D), lambda i: (0, 0))
    return pl.pallas_call(
        functools.partial(_mlp_body, relu_out=relu_out),
        grid=(N // BR,),
        in_specs=[row_spec, part_spec, w_spec, b_spec, w_spec, b_spec],
        out_specs=row_spec,
        out_shape=jax.ShapeDtypeStruct((N, D), jnp.float32),
    )(x, parts, Wa, ba.reshape(1, D), Wb, bb.reshape(1, D))


def kernel(x, edge_index, W1, b1, W2, b2, W3, b3, W4, b4):
    src = edge_index[0].astype(jnp.int32)
    dst = edge_index[1].astype(jnp.int32)
    p1 = _sc_agg(x, src, dst)
    h = _mlp(x, p1, W1, b1, W2, b2, relu_out=True)
    p2 = _sc_agg(h, src, dst)
    return _mlp(h, p2, W3, b3, W4, b4, relu_out=False)
